# baseline (speedup 1.0000x reference)
import functools

import numpy as np
import jax
import jax.numpy as jnp
from jax import lax
from jax.experimental import pallas as pl
from jax.experimental.pallas import tpu as pltpu
from jax.experimental.pallas import tpu_sc as plsc

_SEQ = 200
_D = 32
_BATCH = 4096
_NW = 32
_BBLK = 512
_NBLK = _BATCH // _BBLK
_NITEM = _SEQ * _NBLK
_IPW = _NITEM // _NW
_NPAIR = _IPW // 2
_NG = _BBLK // 128
_TP = 131


def _positional_embedding():
    pos = np.arange(_SEQ, dtype=np.float32)[:, None]
    exp_sin = np.arange(0, _D, 2, dtype=np.float32) / _D * 2.0
    exp_cos = np.arange(1, _D + 1, 2, dtype=np.float32) / _D * 2.0
    sins = np.sin(pos / np.power(10000.0, exp_sin))
    coss = np.cos(pos / np.power(10000.0, exp_cos))
    pe = np.stack([sins, coss], axis=2).reshape(_SEQ, _D)
    return jnp.asarray(pe, dtype=jnp.float32)


def _body(
    idx_hbm, pe_hbm, table_hbm, out_hbm,
    idx_a, idx_b, pe_v, g_a, g_b, t_a, t_b, sem_a, sem_b, osem_a, osem_b,
):
    wid = lax.axis_index("s") * 2 + lax.axis_index("c")
    pltpu.sync_copy(pe_hbm, pe_v)

    def fire(item, idx_v, g_v, sem):
        s = item // _NBLK
        j = item % _NBLK
        rbase = (s >> 3) * 256 + (s & 7)
        for r in range(_NG):
            pltpu.make_async_copy(
                idx_hbm.at[pl.ds(rbase + (j * _NG + r) * 8, 1)],
                idx_v.at[pl.ds(r, 1)],
                sem,
            ).start()
        for r in range(_NG):
            pltpu.make_async_copy(
                idx_hbm.at[pl.ds(rbase + (j * _NG + r) * 8, 1)],
                idx_v.at[pl.ds(r, 1)],
                sem,
            ).wait()
        for r in range(_NG):
            pltpu.make_async_copy(
                table_hbm.at[idx_v.at[r]],
                g_v.at[pl.ds(r * 128, 128)],
                sem,
            ).start()

    def drain(idx_v, g_v, sem):
        for r in range(_NG):
            pltpu.make_async_copy(
                table_hbm.at[idx_v.at[r]],
                g_v.at[pl.ds(r * 128, 128)],
                sem,
            ).wait()

    def process(item, g_v, t_v, osem, first):
        @pl.when(jnp.logical_not(first))
        def _():
            for _i in range(16):
                pltpu.make_async_copy(
                    t_v.at[0, :, pl.ds(0, 128)], out_hbm.at[0, 0, 0], osem
                ).wait()

        s = item // _NBLK
        j = item % _NBLK

        def row_body(i, c2):
            lanes = jax.lax.iota(jnp.int32, 16)
            base = (lanes >> 3) * 4
            dinv = lanes & 7
            p0 = pe_v[s, pl.ds(0, 16)]
            p1 = pe_v[s, pl.ds(16, 16)]
            for u in range(16):
                b = i * 16 + u
                chunk0 = base + jnp.broadcast_to(b >> 7, (16,))
                binv = jnp.broadcast_to(b & 127, (16,))
                plsc.store_scatter(t_v, [chunk0, dinv, binv], g_v[b, pl.ds(0, 16)] + p0)
                plsc.store_scatter(t_v, [chunk0 + 8, dinv, binv], g_v[b, pl.ds(16, 16)] + p1)
            return c2

        lax.fori_loop(0, _BBLK // 16, row_body, 0, unroll=False)

        for dt in range(4):
            for r in range(_NG):
                pltpu.make_async_copy(
                    t_v.at[dt * 4 + r, :, pl.ds(0, 128)],
                    out_hbm.at[s, dt, j * _NG + r],
                    osem,
                ).start()

    fire(wid, idx_a, g_a, sem_a)

    def pair_body(p, carry):
        e = wid + 64 * p
        fire(e + _NW, idx_b, g_b, sem_b)
        drain(idx_a, g_a, sem_a)
        process(e, g_a, t_a, osem_a, p == 0)
        fire(e + 2 * _NW, idx_a, g_a, sem_a)
        drain(idx_b, g_b, sem_b)
        process(e + _NW, g_b, t_b, osem_b, p == 0)
        return carry

    lax.fori_loop(0, _NPAIR - 1, pair_body, 0, unroll=False)

    e = wid + 64 * (_NPAIR - 1)
    fire(e + _NW, idx_b, g_b, sem_b)
    drain(idx_a, g_a, sem_a)
    process(e, g_a, t_a, osem_a, jnp.bool_(False))
    drain(idx_b, g_b, sem_b)
    process(e + _NW, g_b, t_b, osem_b, jnp.bool_(False))

    for t_v, osem in ((t_a, osem_a), (t_b, osem_b)):
        for _i in range(16):
            pltpu.make_async_copy(
                t_v.at[0, :, pl.ds(0, 128)], out_hbm.at[0, 0, 0], osem
            ).wait()


@jax.jit
def _embed(idx_grouped, pe, token_table):
    mesh = plsc.VectorSubcoreMesh(core_axis_name="c", subcore_axis_name="s")
    run = functools.partial(
        pl.kernel,
        mesh=mesh,
        out_type=jax.ShapeDtypeStruct((_SEQ, 4, 32, 8, 128), jnp.float32),
        scratch_types=[
            pltpu.VMEM((_NG, 128), jnp.int32),
            pltpu.VMEM((_NG, 128), jnp.int32),
            pltpu.VMEM((_SEQ, _D), jnp.float32),
            pltpu.VMEM((_BBLK, _D), jnp.float32),
            pltpu.VMEM((_BBLK, _D), jnp.float32),
            pltpu.VMEM((16, 8, _TP), jnp.float32),
            pltpu.VMEM((16, 8, _TP), jnp.float32),
            pltpu.SemaphoreType.DMA,
            pltpu.SemaphoreType.DMA,
            pltpu.SemaphoreType.DMA,
            pltpu.SemaphoreType.DMA,
        ],
        compiler_params=pltpu.CompilerParams(
            use_tc_tiling_on_sc=False, needs_layout_passes=False
        ),
    )(_body)
    return run(idx_grouped, pe, token_table)


def kernel(sequence, token_table):
    idx_grouped = (
        jnp.transpose(sequence)
        .reshape(_SEQ // 8, 8, _BATCH // 128, 128)
        .transpose(0, 2, 1, 3)
        .reshape(_SEQ * _BATCH // 128, 128)
    )
    pe = _positional_embedding()
    f5 = _embed(idx_grouped, pe, token_table)
    return f5.transpose(2, 4, 0, 1, 3).reshape(_BATCH, _SEQ, _D)


# speedup vs baseline: 1.0036x; 1.0036x over previous
import functools

import numpy as np
import jax
import jax.numpy as jnp
from jax import lax
from jax.experimental import pallas as pl
from jax.experimental.pallas import tpu as pltpu
from jax.experimental.pallas import tpu_sc as plsc

_SEQ = 200
_D = 32
_BATCH = 4096
_NW = 32
_BBLK = 512
_NBLK = _BATCH // _BBLK
_NITEM = _SEQ * _NBLK
_IPW = _NITEM // _NW
_NPAIR = _IPW // 2
_NG = _BBLK // 128
_TP = 131


def _positional_embedding():
    pos = np.arange(_SEQ, dtype=np.float32)[:, None]
    exp_sin = np.arange(0, _D, 2, dtype=np.float32) / _D * 2.0
    exp_cos = np.arange(1, _D + 1, 2, dtype=np.float32) / _D * 2.0
    sins = np.sin(pos / np.power(10000.0, exp_sin))
    coss = np.cos(pos / np.power(10000.0, exp_cos))
    pe = np.stack([sins, coss], axis=2).reshape(_SEQ, _D)
    return jnp.asarray(pe, dtype=jnp.float32)


def _body(
    idx_hbm, pe_hbm, table_hbm, out_hbm,
    idx_a, idx_b, pe_v, g_a, g_b, t_a, t_b, sem_a, sem_b, osem_a, osem_b,
):
    wid = lax.axis_index("s") * 2 + lax.axis_index("c")
    pltpu.sync_copy(pe_hbm, pe_v)

    def fire(item, idx_v, g_v, sem):
        s = item // _NBLK
        j = item % _NBLK
        rbase = (s >> 3) * 256 + (s & 7)
        for r in range(_NG):
            pltpu.make_async_copy(
                idx_hbm.at[pl.ds(rbase + (j * _NG + r) * 8, 1)],
                idx_v.at[pl.ds(r, 1)],
                sem,
            ).start()
        for r in range(_NG):
            pltpu.make_async_copy(
                idx_hbm.at[pl.ds(rbase + (j * _NG + r) * 8, 1)],
                idx_v.at[pl.ds(r, 1)],
                sem,
            ).wait()
        for r in range(_NG):
            pltpu.make_async_copy(
                table_hbm.at[idx_v.at[r]],
                g_v.at[pl.ds(r * 128, 128)],
                sem,
            ).start()

    def drain(idx_v, g_v, sem):
        for r in range(_NG):
            pltpu.make_async_copy(
                table_hbm.at[idx_v.at[r]],
                g_v.at[pl.ds(r * 128, 128)],
                sem,
            ).wait()

    def process(item, g_v, t_v, osem, first):
        @pl.when(jnp.logical_not(first))
        def _():
            for _i in range(16):
                pltpu.make_async_copy(
                    t_v.at[0, :, pl.ds(0, 128)], out_hbm.at[0, 0, 0], osem
                ).wait()

        s = item // _NBLK
        j = item % _NBLK

        def row_body(i, c2):
            lanes = jax.lax.iota(jnp.int32, 16)
            base = (lanes >> 3) * 4
            dinv = lanes & 7
            p0 = pe_v[s, pl.ds(0, 16)]
            p1 = pe_v[s, pl.ds(16, 16)]
            for u in range(8):
                b = i * 8 + u
                chunk0 = base + jnp.broadcast_to(b >> 7, (16,))
                binv = jnp.broadcast_to(b & 127, (16,))
                plsc.store_scatter(t_v, [chunk0, dinv, binv], g_v[b, pl.ds(0, 16)] + p0)
                plsc.store_scatter(t_v, [chunk0 + 8, dinv, binv], g_v[b, pl.ds(16, 16)] + p1)
            return c2

        lax.fori_loop(0, _BBLK // 8, row_body, 0, unroll=False)

        for dt in range(4):
            for r in range(_NG):
                pltpu.make_async_copy(
                    t_v.at[dt * 4 + r, :, pl.ds(0, 128)],
                    out_hbm.at[s, dt, j * _NG + r],
                    osem,
                ).start()

    fire(wid, idx_a, g_a, sem_a)

    def pair_body(p, carry):
        e = wid + 64 * p
        fire(e + _NW, idx_b, g_b, sem_b)
        drain(idx_a, g_a, sem_a)
        process(e, g_a, t_a, osem_a, p == 0)
        fire(e + 2 * _NW, idx_a, g_a, sem_a)
        drain(idx_b, g_b, sem_b)
        process(e + _NW, g_b, t_b, osem_b, p == 0)
        return carry

    lax.fori_loop(0, _NPAIR - 1, pair_body, 0, unroll=False)

    e = wid + 64 * (_NPAIR - 1)
    fire(e + _NW, idx_b, g_b, sem_b)
    drain(idx_a, g_a, sem_a)
    process(e, g_a, t_a, osem_a, jnp.bool_(False))
    drain(idx_b, g_b, sem_b)
    process(e + _NW, g_b, t_b, osem_b, jnp.bool_(False))

    for t_v, osem in ((t_a, osem_a), (t_b, osem_b)):
        for _i in range(16):
            pltpu.make_async_copy(
                t_v.at[0, :, pl.ds(0, 128)], out_hbm.at[0, 0, 0], osem
            ).wait()


@jax.jit
def _embed(idx_grouped, pe, token_table):
    mesh = plsc.VectorSubcoreMesh(core_axis_name="c", subcore_axis_name="s")
    run = functools.partial(
        pl.kernel,
        mesh=mesh,
        out_type=jax.ShapeDtypeStruct((_SEQ, 4, 32, 8, 128), jnp.float32),
        scratch_types=[
            pltpu.VMEM((_NG, 128), jnp.int32),
            pltpu.VMEM((_NG, 128), jnp.int32),
            pltpu.VMEM((_SEQ, _D), jnp.float32),
            pltpu.VMEM((_BBLK, _D), jnp.float32),
            pltpu.VMEM((_BBLK, _D), jnp.float32),
            pltpu.VMEM((16, 8, _TP), jnp.float32),
            pltpu.VMEM((16, 8, _TP), jnp.float32),
            pltpu.SemaphoreType.DMA,
            pltpu.SemaphoreType.DMA,
            pltpu.SemaphoreType.DMA,
            pltpu.SemaphoreType.DMA,
        ],
        compiler_params=pltpu.CompilerParams(
            use_tc_tiling_on_sc=False, needs_layout_passes=False
        ),
    )(_body)
    return run(idx_grouped, pe, token_table)


def kernel(sequence, token_table):
    idx_grouped = (
        jnp.transpose(sequence)
        .reshape(_SEQ // 8, 8, _BATCH // 128, 128)
        .transpose(0, 2, 1, 3)
        .reshape(_SEQ * _BATCH // 128, 128)
    )
    pe = _positional_embedding()
    f5 = _embed(idx_grouped, pe, token_table)
    return f5.transpose(2, 4, 0, 1, 3).reshape(_BATCH, _SEQ, _D)
